# baseline (speedup 1.0000x reference)
import functools

import jax
import jax.numpy as jnp
from jax.experimental import pallas as pl
from jax.experimental.pallas import tpu as pltpu

N = 10000
F = 256


def _mm_kernel(x_ref, w_ref, o_ref):
    o_ref[...] = jnp.dot(x_ref[...], w_ref[...],
                         preferred_element_type=jnp.float32)


def _feat_mm(x, w):
    br = 1000
    return pl.pallas_call(
        _mm_kernel,
        grid=(N // br,),
        in_specs=[
            pl.BlockSpec((br, F), lambda i: (i, 0)),
            pl.BlockSpec((F, F), lambda i: (0, 0)),
        ],
        out_specs=pl.BlockSpec((br, F), lambda i: (i, 0)),
        out_shape=jax.ShapeDtypeStruct((N, F), jnp.float32),
    )(x, w)


def _agg_kernel(adj_ref, y_ref, b_ref, o_ref, *, relu):
    acc = jnp.dot(adj_ref[...], y_ref[...],
                  preferred_element_type=jnp.float32)
    acc = acc + b_ref[...]
    if relu:
        acc = jnp.maximum(acc, 0.0)
    o_ref[...] = acc


def _aggregate(adj, y, b, relu):
    br = 400
    return pl.pallas_call(
        functools.partial(_agg_kernel, relu=relu),
        grid=(N // br,),
        in_specs=[
            pl.BlockSpec((br, N), lambda i: (i, 0)),
            pl.BlockSpec((N, F), lambda i: (0, 0)),
            pl.BlockSpec((1, F), lambda i: (0, 0)),
        ],
        out_specs=pl.BlockSpec((br, F), lambda i: (i, 0)),
        out_shape=jax.ShapeDtypeStruct((N, F), jnp.float32),
        compiler_params=pltpu.CompilerParams(
            dimension_semantics=("arbitrary",),
        ),
    )(adj, y, b)


def kernel(x, adj, W1, b1, W2, b2):
    h = _aggregate(adj, _feat_mm(x, W1), b1.reshape(1, F), relu=True)
    out = _aggregate(adj, _feat_mm(h, W2), b2.reshape(1, F), relu=False)
    return out
